# baseline (speedup 1.0000x reference)
import functools

import jax
import jax.numpy as jnp
from jax import lax
from jax.experimental import pallas as pl
from jax.experimental.pallas import tpu as pltpu
from jax.experimental.pallas import tpu_sc as plsc

N_USERS = 25000
N_ITEMS = 25000
DIM = 64
HDIM = DIM // 2
N_LAYERS = 3
BATCH = 16384

HALF = 25088
NTOT = 2 * HALF
ROWPT = NTOT // 16

E = 800000
CH = 128
NCH = 8
BLK = NCH * CH
NBLK = 49
NSUP = NBLK
EPT = NBLK * BLK
EPAD = 16 * EPT

PGROWS = 2 * BATCH // CH
PGPW = PGROWS // 32

_mesh = plsc.VectorSubcoreMesh(core_axis_name="c", subcore_axis_name="s")

_GDN = lax.GatherDimensionNumbers(
    offset_dims=(), collapsed_slice_dims=(0,), start_index_map=(0,))


def _bcast16(vec, e):
    idx = jnp.full((16, 1), e, jnp.int32)
    return lax.gather(vec, idx, _GDN, (1,),
                      mode=lax.GatherScatterMode.PROMISE_IN_BOUNDS)


RING = 4


_XSD = jax.ShapeDtypeStruct((N_LAYERS + 1, 2 * NTOT, HDIM), jnp.float32)


@functools.partial(
    pl.kernel,
    mesh=_mesh,
    compiler_params=pltpu.CompilerParams(use_tc_tiling_on_sc=False),
    out_type=_XSD,
    scratch_types=[
        pltpu.VMEM((2, NCH, CH), jnp.int32),
        pltpu.VMEM((2, NCH, CH), jnp.int32),
        pltpu.VMEM((2, BLK), jnp.float32),
        pltpu.VMEM((RING, CH, HDIM), jnp.float32),
        pltpu.VMEM_SHARED((NTOT, HDIM), jnp.float32),
        pltpu.SemaphoreType.DMA,
        pltpu.SemaphoreType.DMA,
        pltpu.SemaphoreType.DMA,
        pltpu.SemaphoreType.DMA,
        pltpu.SemaphoreType.DMA,
        pltpu.SemaphoreType.DMA,
        pltpu.SemaphoreType.DMA,
        pltpu.SemaphoreType.DMA,
        pltpu.SemaphoreType.DMA,
    ],
)
def _layers(x0, sidx, dsti, wgt, zrows, xs, idxs, idxd, w2, rows,
            acc, isem, g0, g1, g2, g3, s0, s1, s2, s3):
    c = lax.axis_index("c")
    s = lax.axis_index("s")
    gsem = [g0, g1, g2, g3]
    ssem = [s0, s1, s2, s3]

    def _pass(l):
        x_in = xs.at[l]
        x_out = xs.at[l + 1]
        pltpu.sync_copy(zrows, acc.at[pl.ds(s * ROWPT, ROWPT)])
        plsc.subcore_barrier()

        def _fire_idx(u, p):
            row0 = s * (EPT // CH) + u * NCH
            pltpu.async_copy(sidx.at[c, pl.ds(row0, NCH)], idxs.at[p], isem)
            pltpu.async_copy(dsti.at[pl.ds(row0, NCH)], idxd.at[p], isem)
            pltpu.async_copy(wgt.at[pl.ds(s * EPT + u * BLK, BLK)],
                             w2.at[p], isem)

        def _drain_idx(u, p):
            row0 = s * (EPT // CH) + u * NCH
            pltpu.make_async_copy(sidx.at[c, pl.ds(row0, NCH)], idxs.at[p],
                                  isem).wait()
            pltpu.make_async_copy(dsti.at[pl.ds(row0, NCH)], idxd.at[p],
                                  isem).wait()
            pltpu.make_async_copy(wgt.at[pl.ds(s * EPT + u * BLK, BLK)],
                                  w2.at[p], isem).wait()

        def _fire_gather(p, j, q):
            pltpu.async_copy(x_in.at[idxs.at[p, j]], rows.at[q], gsem[q])

        def _proc(p, j):
            q = j % RING
            pltpu.make_async_copy(x_in.at[idxs.at[p, j]], rows.at[q],
                                  gsem[q]).wait()

            def _grp(g, carry):
                wv = w2[p, pl.ds(j * CH + g * 16, 16)]
                for e in range(16):
                    wb = _bcast16(wv, e)
                    r = g * 16 + e
                    for k in range(HDIM // 16):
                        sl = pl.ds(k * 16, 16)
                        rows[q, r, sl] = rows[q, r, sl] * wb
                return carry

            lax.fori_loop(0, CH // 16, _grp, 0)
            pltpu.async_copy(rows.at[q], acc.at[idxd.at[p, j]], ssem[q],
                             add=True)

        def _drain_scat(p, j):
            q = j % RING
            pltpu.make_async_copy(rows.at[q], acc.at[idxd.at[p, j]],
                                  ssem[q]).wait()

        def _super(u, p, first=False, last=False):
            for j in range(NCH):
                if j == 1 and not last:
                    _fire_idx(u + 1, p ^ 1)
                if j == 5 and not last:
                    _drain_idx(u + 1, p ^ 1)
                if j >= 2:
                    _drain_scat(p, j - 2)
                elif not first:
                    _drain_scat(p ^ 1, j + 6)
                if j < NCH - 2:
                    _fire_gather(p, j + 2, (j + 2) % RING)
                elif not last:
                    _fire_gather(p ^ 1, j + 2 - NCH, (j + 2) % RING)
                _proc(p, j)

        _fire_idx(0, 0)
        _drain_idx(0, 0)
        _fire_gather(0, 0, 0)
        _fire_gather(0, 1, 1)
        _super(0, 0, first=True)

        def _sup_pair(k, carry):
            u = 2 * k + 1
            _super(u, 1)
            _super(u + 1, 0)
            return carry

        lax.fori_loop(0, (NSUP - 3) // 2, _sup_pair, 0)

        _super(NSUP - 2, 1)
        _super(NSUP - 1, 0, last=True)

        _drain_scat(0, 6)
        _drain_scat(0, 7)

        plsc.subcore_barrier()

        pltpu.sync_copy(acc.at[pl.ds(s * ROWPT, ROWPT)],
                        x_out.at[pl.ds(c * NTOT + s * ROWPT, ROWPT)])
        plsc.subcore_barrier()

    w = c * 16 + s
    pltpu.sync_copy(x0.at[pl.ds(w * ROWPT, ROWPT)],
                    xs.at[0, pl.ds(w * ROWPT, ROWPT)])
    plsc.subcore_barrier()

    def _layer(l, carry):
        _pass(l)
        return carry

    lax.fori_loop(0, N_LAYERS, _layer, 0)


@functools.partial(
    pl.kernel,
    mesh=_mesh,
    compiler_params=pltpu.CompilerParams(use_tc_tiling_on_sc=False),
    out_type=jax.ShapeDtypeStruct((2 * BATCH, DIM), jnp.float32),
    scratch_types=[
        pltpu.VMEM((PGPW, CH), jnp.int32),
        pltpu.VMEM((CH, DIM), jnp.float32),
        pltpu.SemaphoreType.DMA,
    ],
)
def _pair_gather(h, pidx, out, idx_v, rows_v, sem):
    c = lax.axis_index("c")
    s = lax.axis_index("s")
    wid = s * 2 + c

    pltpu.sync_copy(pidx.at[pl.ds(wid * PGPW, PGPW)], idx_v)

    def _row(j, carry):
        pltpu.async_copy(h.at[idx_v.at[j]], rows_v, sem).wait()
        pltpu.sync_copy(rows_v, out.at[pl.ds((wid * PGPW + j) * CH, CH)])
        return carry

    lax.fori_loop(0, PGPW, _row, 0)


def _hsum_body(a0, a1, a2, a3, b0, b1, b2, b3, o_ref):
    o_ref[:, :HDIM] = (a0[0] + a1[0] + a2[0] + a3[0]) * 0.25
    o_ref[:, HDIM:] = (b0[0] + b1[0] + b2[0] + b3[0]) * 0.25


_HB = 1568
_hsum = pl.pallas_call(
    _hsum_body,
    grid=(NTOT // _HB,),
    in_specs=[pl.BlockSpec((1, _HB, HDIM), functools.partial(
        lambda l, i: (l, i, 0), l)) for l in range(4)]
    + [pl.BlockSpec((1, _HB, HDIM), functools.partial(
        lambda l, i: (l, i + NTOT // _HB, 0), l)) for l in range(4)],
    out_specs=pl.BlockSpec((_HB, DIM), lambda i: (i, 0)),
    out_shape=jax.ShapeDtypeStruct((NTOT, DIM), jnp.float32),
)


def _dot_body(u_ref, v_ref, o_ref):
    o_ref[...] = jnp.sum(u_ref[...] * v_ref[...], axis=1)


_dot = pl.pallas_call(
    _dot_body,
    grid=(BATCH // 2048,),
    in_specs=[
        pl.BlockSpec((2048, DIM), lambda i: (i, 0)),
        pl.BlockSpec((2048, DIM), lambda i: (i + BATCH // 2048, 0)),
    ],
    out_specs=pl.BlockSpec((2048,), lambda i: (i,)),
    out_shape=jax.ShapeDtypeStruct((BATCH,), jnp.float32),
)


def kernel(users, items, user_emb, item_emb, edge_index, edge_weight):
    src = edge_index[0].astype(jnp.int32)
    dst = edge_index[1].astype(jnp.int32)

    src_p = jnp.zeros((EPAD,), jnp.int32).at[:E].set(src)
    dst_p = jnp.zeros((EPAD,), jnp.int32).at[:E].set(dst)
    w_p = jnp.zeros((EPAD,), jnp.float32).at[:E].set(edge_weight)

    s_row = jnp.where(src_p >= N_USERS, src_p + (HALF - N_USERS), src_p)
    sidx = jnp.stack([s_row, s_row + NTOT]).reshape(2, EPAD // CH, CH)
    d_row = jnp.where(dst_p >= N_USERS, dst_p + (HALF - N_USERS), dst_p)
    dsti = d_row.reshape(EPAD // CH, CH)

    x0 = jnp.zeros((2 * NTOT, HDIM), jnp.float32)
    x0 = lax.dynamic_update_slice(x0, user_emb[:, :HDIM], (0, 0))
    x0 = lax.dynamic_update_slice(x0, item_emb[:, :HDIM], (HALF, 0))
    x0 = lax.dynamic_update_slice(x0, user_emb[:, HDIM:], (NTOT, 0))
    x0 = lax.dynamic_update_slice(x0, item_emb[:, HDIM:], (NTOT + HALF, 0))

    zrows = jnp.zeros((ROWPT, HDIM), jnp.float32)

    xs = _layers(x0, sidx, dsti, w_p, zrows)

    h = _hsum(*([xs] * 8))

    pidx = jnp.concatenate([users.astype(jnp.int32),
                            items.astype(jnp.int32) + HALF])
    pidx = pidx.reshape(PGROWS, CH)
    uv = _pair_gather(h, pidx)
    return _dot(uv, uv)


# speedup vs baseline: 1.4350x; 1.4350x over previous
import functools

import jax
import jax.numpy as jnp
from jax import lax
from jax.experimental import pallas as pl
from jax.experimental.pallas import tpu as pltpu
from jax.experimental.pallas import tpu_sc as plsc

N_USERS = 25000
N_ITEMS = 25000
DIM = 64
HDIM = DIM // 2
N_LAYERS = 3
BATCH = 16384

HALF = 25088
NTOT = 2 * HALF
ROWPT = NTOT // 16

E = 800000
CH = 128
NCH = 8
BLK = NCH * CH
NBLK = 49
NSUP = NBLK
EPT = NBLK * BLK
EPAD = 16 * EPT

PGROWS = 2 * BATCH // CH
PGPW = PGROWS // 32

_mesh = plsc.VectorSubcoreMesh(core_axis_name="c", subcore_axis_name="s")

_GDN = lax.GatherDimensionNumbers(
    offset_dims=(), collapsed_slice_dims=(0,), start_index_map=(0,))


def _bcast16(vec, e):
    idx = jnp.full((16, 1), e, jnp.int32)
    return lax.gather(vec, idx, _GDN, (1,),
                      mode=lax.GatherScatterMode.PROMISE_IN_BOUNDS)


RING = 4


_XSD = jax.ShapeDtypeStruct((2 * NTOT, HDIM), jnp.float32)


@functools.partial(
    pl.kernel,
    mesh=_mesh,
    compiler_params=pltpu.CompilerParams(use_tc_tiling_on_sc=False),
    out_type=_XSD,
    scratch_types=[
        pltpu.VMEM((2, NCH, CH), jnp.int32),
        pltpu.VMEM((2, NCH, CH), jnp.int32),
        pltpu.VMEM((2, BLK), jnp.float32),
        pltpu.VMEM((RING, CH, HDIM), jnp.float32),
        pltpu.VMEM_SHARED((NTOT, HDIM), jnp.float32),
        pltpu.SemaphoreType.DMA,
        pltpu.SemaphoreType.DMA,
        pltpu.SemaphoreType.DMA,
        pltpu.SemaphoreType.DMA,
        pltpu.SemaphoreType.DMA,
        pltpu.SemaphoreType.DMA,
        pltpu.SemaphoreType.DMA,
        pltpu.SemaphoreType.DMA,
        pltpu.SemaphoreType.DMA,
    ],
)
def _propagate(x_in, sidx, dsti, wgt, zrows, x_out, idxs, idxd, w2, rows,
               acc, isem, g0, g1, g2, g3, s0, s1, s2, s3):
    c = lax.axis_index("c")
    s = lax.axis_index("s")
    gsem = [g0, g1, g2, g3]
    ssem = [s0, s1, s2, s3]

    def _pass():
        pltpu.sync_copy(zrows, acc.at[pl.ds(s * ROWPT, ROWPT)])
        plsc.subcore_barrier()

        def _fire_idx(u, p):
            row0 = s * (EPT // CH) + u * NCH
            pltpu.async_copy(sidx.at[c, pl.ds(row0, NCH)], idxs.at[p], isem)
            pltpu.async_copy(dsti.at[pl.ds(row0, NCH)], idxd.at[p], isem)
            pltpu.async_copy(wgt.at[pl.ds(s * EPT + u * BLK, BLK)],
                             w2.at[p], isem)

        def _drain_idx(u, p):
            row0 = s * (EPT // CH) + u * NCH
            pltpu.make_async_copy(sidx.at[c, pl.ds(row0, NCH)], idxs.at[p],
                                  isem).wait()
            pltpu.make_async_copy(dsti.at[pl.ds(row0, NCH)], idxd.at[p],
                                  isem).wait()
            pltpu.make_async_copy(wgt.at[pl.ds(s * EPT + u * BLK, BLK)],
                                  w2.at[p], isem).wait()

        def _fire_gather(p, j, q):
            pltpu.async_copy(x_in.at[idxs.at[p, j]], rows.at[q], gsem[q])

        def _proc(p, j):
            q = j % RING
            pltpu.make_async_copy(x_in.at[idxs.at[p, j]], rows.at[q],
                                  gsem[q]).wait()

            def _grp(g, carry):
                wv = w2[p, pl.ds(j * CH + g * 16, 16)]
                for e in range(16):
                    wb = _bcast16(wv, e)
                    r = g * 16 + e
                    for k in range(HDIM // 16):
                        sl = pl.ds(k * 16, 16)
                        rows[q, r, sl] = rows[q, r, sl] * wb
                return carry

            lax.fori_loop(0, CH // 16, _grp, 0)
            pltpu.async_copy(rows.at[q], acc.at[idxd.at[p, j]], ssem[q],
                             add=True)

        def _drain_scat(p, j):
            q = j % RING
            pltpu.make_async_copy(rows.at[q], acc.at[idxd.at[p, j]],
                                  ssem[q]).wait()

        def _super(u, p, first=False, last=False):
            for j in range(NCH):
                if j == 1 and not last:
                    _fire_idx(u + 1, p ^ 1)
                if j == 5 and not last:
                    _drain_idx(u + 1, p ^ 1)
                if j >= 2:
                    _drain_scat(p, j - 2)
                elif not first:
                    _drain_scat(p ^ 1, j + 6)
                if j < NCH - 2:
                    _fire_gather(p, j + 2, (j + 2) % RING)
                elif not last:
                    _fire_gather(p ^ 1, j + 2 - NCH, (j + 2) % RING)
                _proc(p, j)

        _fire_idx(0, 0)
        _drain_idx(0, 0)
        _fire_gather(0, 0, 0)
        _fire_gather(0, 1, 1)
        _super(0, 0, first=True)

        def _sup_pair(k, carry):
            u = 2 * k + 1
            _super(u, 1)
            _super(u + 1, 0)
            return carry

        lax.fori_loop(0, (NSUP - 3) // 2, _sup_pair, 0)

        _super(NSUP - 2, 1)
        _super(NSUP - 1, 0, last=True)

        _drain_scat(0, 6)
        _drain_scat(0, 7)

        plsc.subcore_barrier()

        pltpu.sync_copy(acc.at[pl.ds(s * ROWPT, ROWPT)],
                        x_out.at[pl.ds(c * NTOT + s * ROWPT, ROWPT)])

    _pass()


@functools.partial(
    pl.kernel,
    mesh=_mesh,
    compiler_params=pltpu.CompilerParams(use_tc_tiling_on_sc=False),
    out_type=jax.ShapeDtypeStruct((2 * BATCH, DIM), jnp.float32),
    scratch_types=[
        pltpu.VMEM((PGPW, CH), jnp.int32),
        pltpu.VMEM((CH, DIM), jnp.float32),
        pltpu.SemaphoreType.DMA,
    ],
)
def _pair_gather(h, pidx, out, idx_v, rows_v, sem):
    c = lax.axis_index("c")
    s = lax.axis_index("s")
    wid = s * 2 + c

    pltpu.sync_copy(pidx.at[pl.ds(wid * PGPW, PGPW)], idx_v)

    def _row(j, carry):
        pltpu.async_copy(h.at[idx_v.at[j]], rows_v, sem).wait()
        pltpu.sync_copy(rows_v, out.at[pl.ds((wid * PGPW + j) * CH, CH)])
        return carry

    lax.fori_loop(0, PGPW, _row, 0)


def _hsum_body(a0, a1, a2, a3, b0, b1, b2, b3, o_ref):
    o_ref[:, :HDIM] = (a0[...] + a1[...] + a2[...] + a3[...]) * 0.25
    o_ref[:, HDIM:] = (b0[...] + b1[...] + b2[...] + b3[...]) * 0.25


_HB = 1568
_hsum = pl.pallas_call(
    _hsum_body,
    grid=(NTOT // _HB,),
    in_specs=[pl.BlockSpec((_HB, HDIM), lambda i: (i, 0))] * 4
    + [pl.BlockSpec((_HB, HDIM), lambda i: (i + NTOT // _HB, 0))] * 4,
    out_specs=pl.BlockSpec((_HB, DIM), lambda i: (i, 0)),
    out_shape=jax.ShapeDtypeStruct((NTOT, DIM), jnp.float32),
)


def _dot_body(u_ref, v_ref, o_ref):
    o_ref[...] = jnp.sum(u_ref[...] * v_ref[...], axis=1)


_dot = pl.pallas_call(
    _dot_body,
    grid=(BATCH // 2048,),
    in_specs=[
        pl.BlockSpec((2048, DIM), lambda i: (i, 0)),
        pl.BlockSpec((2048, DIM), lambda i: (i + BATCH // 2048, 0)),
    ],
    out_specs=pl.BlockSpec((2048,), lambda i: (i,)),
    out_shape=jax.ShapeDtypeStruct((BATCH,), jnp.float32),
)


def kernel(users, items, user_emb, item_emb, edge_index, edge_weight):
    src = edge_index[0].astype(jnp.int32)
    dst = edge_index[1].astype(jnp.int32)

    src_p = jnp.zeros((EPAD,), jnp.int32).at[:E].set(src)
    dst_p = jnp.zeros((EPAD,), jnp.int32).at[:E].set(dst)
    w_p = jnp.zeros((EPAD,), jnp.float32).at[:E].set(edge_weight)

    s_row = jnp.where(src_p >= N_USERS, src_p + (HALF - N_USERS), src_p)
    sidx = jnp.stack([s_row, s_row + NTOT]).reshape(2, EPAD // CH, CH)
    d_row = jnp.where(dst_p >= N_USERS, dst_p + (HALF - N_USERS), dst_p)
    dsti = d_row.reshape(EPAD // CH, CH)

    x0 = jnp.zeros((2 * NTOT, HDIM), jnp.float32)
    x0 = lax.dynamic_update_slice(x0, user_emb[:, :HDIM], (0, 0))
    x0 = lax.dynamic_update_slice(x0, item_emb[:, :HDIM], (HALF, 0))
    x0 = lax.dynamic_update_slice(x0, user_emb[:, HDIM:], (NTOT, 0))
    x0 = lax.dynamic_update_slice(x0, item_emb[:, HDIM:], (NTOT + HALF, 0))

    zrows = jnp.zeros((ROWPT, HDIM), jnp.float32)

    x1 = _propagate(x0, sidx, dsti, w_p, zrows)
    x2 = _propagate(x1, sidx, dsti, w_p, zrows)
    x3 = _propagate(x2, sidx, dsti, w_p, zrows)
    tables = [x0, x1, x2, x3]

    h = _hsum(*tables, *tables)

    pidx = jnp.concatenate([users.astype(jnp.int32),
                            items.astype(jnp.int32) + HALF])
    pidx = pidx.reshape(PGROWS, CH)
    uv = _pair_gather(h, pidx)
    return _dot(uv, uv)
